# Initial kernel scaffold; baseline (speedup 1.0000x reference)
import jax
import jax.numpy as jnp
from jax.experimental import pallas as pl


def kernel(token_ids, emb):
    raise NotImplementedError("write your pallas kernel here")


# speedup vs baseline: 1.8315x; 1.8315x over previous
import functools

import jax
import jax.numpy as jnp
from jax import lax
from jax.experimental import pallas as pl
from jax.experimental.pallas import tpu as pltpu
from jax.experimental.pallas import tpu_sc as plsc

_D = 64
_B = 16384 * 50
_NW = 32
_PER_W = _B // _NW
_CHUNK = 800
_NCHUNK = _PER_W // _CHUNK

_mesh = plsc.VectorSubcoreMesh(core_axis_name="c", subcore_axis_name="s")


@functools.partial(
    pl.kernel,
    mesh=_mesh,
    out_type=jax.ShapeDtypeStruct((_B, _D), jnp.float32),
    scratch_types=[
        pltpu.VMEM((_CHUNK,), jnp.int32),
        pltpu.VMEM((_CHUNK, _D), jnp.float32),
        pltpu.SemaphoreType.DMA,
    ],
    compiler_params=pltpu.CompilerParams(use_tc_tiling_on_sc=False),
)
def _gather_kernel(idx_hbm, table_hbm, out_hbm, idx_v, rows_v, sem):
    wid = lax.axis_index("s") * 2 + lax.axis_index("c")
    base = wid * _PER_W

    def body(g, carry):
        off = base + g * _CHUNK
        pltpu.sync_copy(idx_hbm.at[pl.ds(off, _CHUNK)], idx_v)
        pltpu.async_copy(table_hbm.at[idx_v], rows_v, sem).wait()
        pltpu.sync_copy(rows_v, out_hbm.at[pl.ds(off, _CHUNK)])
        return carry

    lax.fori_loop(0, _NCHUNK, body, 0)


def kernel(token_ids, emb):
    idx = token_ids.reshape(-1).astype(jnp.int32)
    out = _gather_kernel(idx, emb)
    return out.reshape(token_ids.shape + (_D,))


# speedup vs baseline: 1.8711x; 1.0216x over previous
import functools

import jax
import jax.numpy as jnp
from jax import lax
from jax.experimental import pallas as pl
from jax.experimental.pallas import tpu as pltpu
from jax.experimental.pallas import tpu_sc as plsc

_D = 64
_B = 16384 * 50
_NW = 32
_PER_W = _B // _NW
_CHUNK = 800
_NCHUNK = _PER_W // _CHUNK

_mesh = plsc.VectorSubcoreMesh(core_axis_name="c", subcore_axis_name="s")


@functools.partial(
    pl.kernel,
    mesh=_mesh,
    out_type=jax.ShapeDtypeStruct((_B, _D), jnp.float32),
    scratch_types=[
        pltpu.VMEM((_CHUNK,), jnp.int32),
        pltpu.VMEM((_CHUNK,), jnp.int32),
        pltpu.VMEM((_CHUNK, _D), jnp.float32),
        pltpu.VMEM((_CHUNK, _D), jnp.float32),
        pltpu.SemaphoreType.DMA,
        pltpu.SemaphoreType.DMA,
        pltpu.SemaphoreType.DMA,
        pltpu.SemaphoreType.DMA,
        pltpu.SemaphoreType.DMA,
        pltpu.SemaphoreType.DMA,
    ],
    compiler_params=pltpu.CompilerParams(use_tc_tiling_on_sc=False),
)
def _gather_kernel(idx_hbm, table_hbm, out_hbm, idx0, idx1, rows0, rows1,
                   si0, si1, sg0, sg1, ss0, ss1):
    wid = lax.axis_index("s") * 2 + lax.axis_index("c")
    base = wid * _PER_W
    idx_v = (idx0, idx1)
    rows_v = (rows0, rows1)
    sem_i = (si0, si1)
    sem_g = (sg0, sg1)
    sem_s = (ss0, ss1)

    def start_idx(g, b):
        off = base + jnp.minimum(g, _NCHUNK - 1) * _CHUNK
        pltpu.async_copy(idx_hbm.at[pl.ds(off, _CHUNK)], idx_v[b], sem_i[b])

    def wait_idx(b):
        pltpu.make_async_copy(
            idx_hbm.at[pl.ds(base, _CHUNK)], idx_v[b], sem_i[b]).wait()

    def start_gather(b):
        pltpu.async_copy(table_hbm.at[idx_v[b]], rows_v[b], sem_g[b])

    def wait_gather(b):
        pltpu.make_async_copy(
            table_hbm.at[pl.ds(0, _CHUNK)], rows_v[b], sem_g[b]).wait()

    def start_store(g, b):
        off = base + g * _CHUNK
        pltpu.async_copy(rows_v[b], out_hbm.at[pl.ds(off, _CHUNK)], sem_s[b])

    def wait_store(b):
        pltpu.make_async_copy(
            rows_v[b], out_hbm.at[pl.ds(base, _CHUNK)], sem_s[b]).wait()

    pltpu.sync_copy(idx_hbm.at[pl.ds(base, _CHUNK)], idx0)
    start_gather(0)
    start_idx(1, 1)
    wait_gather(0)
    start_store(0, 0)
    wait_idx(1)
    start_gather(1)
    start_idx(2, 0)

    def chunk_body(g, b):
        nb = 1 - b
        wait_gather(b)
        start_store(g, b)
        wait_idx(nb)
        wait_store(nb)
        start_gather(nb)
        start_idx(g + 2, b)

    def body(i, carry):
        g = 1 + 2 * i
        chunk_body(g, 1)
        chunk_body(g + 1, 0)
        return carry

    lax.fori_loop(0, (_NCHUNK - 2) // 2, body, 0)

    last = _NCHUNK - 1
    b = last % 2
    nb = 1 - b
    wait_gather(b)
    start_store(last, b)
    wait_idx(nb)
    wait_store(nb)
    wait_store(b)


def kernel(token_ids, emb):
    idx = token_ids.reshape(-1).astype(jnp.int32)
    out = _gather_kernel(idx, emb)
    return out.reshape(token_ids.shape + (_D,))
